# Initial kernel scaffold; baseline (speedup 1.0000x reference)
import jax
import jax.numpy as jnp
from jax.experimental import pallas as pl


def kernel(x, E):
    raise NotImplementedError("write your pallas kernel here")


# speedup vs baseline: 4.1847x; 4.1847x over previous
import functools

import jax
import jax.numpy as jnp
from jax import lax
from jax.experimental import pallas as pl
from jax.experimental.pallas import tpu as pltpu
from jax.experimental.pallas import tpu_sc as plsc


def _build_sc_embed(R, D, n_workers, P):
    per_w = R // n_workers
    chunk_rows = P // 128
    rows_per_w = per_w // 128
    n_chunks = per_w // P

    mesh = plsc.VectorSubcoreMesh(core_axis_name="c", subcore_axis_name="s")
    nc = 2

    @functools.partial(
        pl.kernel,
        mesh=mesh,
        out_type=jax.ShapeDtypeStruct((R, D), jnp.float32),
        compiler_params=pltpu.CompilerParams(use_tc_tiling_on_sc=False),
        scratch_types=[
            pltpu.VMEM((chunk_rows, 128), jnp.int32),
            pltpu.VMEM((chunk_rows, 128), jnp.int32),
            pltpu.VMEM((P, D), jnp.float32),
            pltpu.VMEM((P, D), jnp.float32),
            pltpu.SemaphoreType.DMA,
        ],
    )
    def body(x0_hbm, x1_hbm, e_hbm, out_hbm, idx0, idx1, buf0, buf1, sem):
        wid = lax.axis_index("s") * nc + lax.axis_index("c")

        def do_chunk(c, carry):
            r0 = wid * rows_per_w + c * chunk_rows
            pltpu.sync_copy(x0_hbm.at[pl.ds(r0, chunk_rows)], idx0)
            pltpu.sync_copy(x1_hbm.at[pl.ds(r0, chunk_rows)], idx1)
            cps = []
            for j in range(chunk_rows):
                cps.append(pltpu.async_copy(
                    e_hbm.at[idx0.at[j]], buf0.at[pl.ds(j * 128, 128)], sem))
                cps.append(pltpu.async_copy(
                    e_hbm.at[idx1.at[j]], buf1.at[pl.ds(j * 128, 128)], sem))
            for cp in cps:
                cp.wait()

            U = 8

            def add_body(i, carry2):
                for u in range(U):
                    p = i * U + u
                    for h in range(D // 16):
                        plsc.addupdate(
                            buf0.at[p, pl.ds(h * 16, 16)],
                            buf1[p, pl.ds(h * 16, 16)],
                        )
                return carry2

            lax.fori_loop(0, P // U, add_body, 0)
            pltpu.sync_copy(buf0, out_hbm.at[pl.ds(wid * per_w + c * P, P)])
            return carry

        lax.fori_loop(0, n_chunks, do_chunk, 0)

    return body


def kernel(x, E):
    B, L, H = x.shape
    V, D = E.shape
    assert H == 2 and D % 16 == 0
    R = B * L
    n_workers = 32
    P = 512
    assert R % (n_workers * P) == 0 and P % 128 == 0

    x = x.astype(jnp.int32)
    xf = x.reshape(R, 2)
    x0 = xf[:, 0].reshape(R // 128, 128)
    x1 = xf[:, 1].reshape(R // 128, 128)

    body = _build_sc_embed(R, D, n_workers, P)
    out = body(x0, x1, E)
    return out.reshape(B, L, D)


# speedup vs baseline: 4.2616x; 1.0184x over previous
import functools

import jax
import jax.numpy as jnp
from jax import lax
from jax.experimental import pallas as pl
from jax.experimental.pallas import tpu as pltpu
from jax.experimental.pallas import tpu_sc as plsc


def _build_sc_embed(R, D, n_workers, P):
    per_w = R // n_workers
    chunk_rows = P // 128
    rows_per_w = per_w // 128
    n_chunks = per_w // P

    mesh = plsc.VectorSubcoreMesh(core_axis_name="c", subcore_axis_name="s")
    nc = 2

    @functools.partial(
        pl.kernel,
        mesh=mesh,
        out_type=jax.ShapeDtypeStruct((R, D), jnp.float32),
        compiler_params=pltpu.CompilerParams(use_tc_tiling_on_sc=False),
        scratch_types=[
            pltpu.VMEM((chunk_rows, 128), jnp.int32),
            pltpu.VMEM((chunk_rows, 128), jnp.int32),
            pltpu.VMEM((P, D), jnp.float32),
            pltpu.VMEM((P, D), jnp.float32),
            pltpu.SemaphoreType.DMA,
        ],
    )
    def body(x0_hbm, x1_hbm, e_hbm, out_hbm, idx0, idx1, buf0, buf1, sem):
        wid = lax.axis_index("s") * nc + lax.axis_index("c")

        def do_chunk(c, carry):
            r0 = wid * rows_per_w + c * chunk_rows
            pltpu.sync_copy(x0_hbm.at[pl.ds(r0, chunk_rows)], idx0)
            pltpu.sync_copy(x1_hbm.at[pl.ds(r0, chunk_rows)], idx1)
            cps = []
            for j in range(chunk_rows):
                cps.append(pltpu.async_copy(
                    e_hbm.at[idx0.at[j]], buf0.at[pl.ds(j * 128, 128)], sem))
            for cp in cps:
                cp.wait()
            cps = []
            for j in range(chunk_rows):
                cps.append(pltpu.async_copy(
                    e_hbm.at[idx1.at[j]], buf0.at[pl.ds(j * 128, 128)], sem,
                    add=True))
            for cp in cps:
                cp.wait()
            pltpu.sync_copy(buf0, out_hbm.at[pl.ds(wid * per_w + c * P, P)])
            return carry

        lax.fori_loop(0, n_chunks, do_chunk, 0)

    return body


def kernel(x, E):
    B, L, H = x.shape
    V, D = E.shape
    assert H == 2 and D % 16 == 0
    R = B * L
    n_workers = 32
    P = 512
    assert R % (n_workers * P) == 0 and P % 128 == 0

    x = x.astype(jnp.int32)
    xf = x.reshape(R, 2)
    x0 = xf[:, 0].reshape(R // 128, 128)
    x1 = xf[:, 1].reshape(R // 128, 128)

    body = _build_sc_embed(R, D, n_workers, P)
    out = body(x0, x1, E)
    return out.reshape(B, L, D)
